# Initial kernel scaffold; baseline (speedup 1.0000x reference)
import jax
import jax.numpy as jnp
from jax.experimental import pallas as pl


def kernel(x, edge_index, edge_weight, gcn_W, gcn_b, fci_W1, fci_b1, fci_W2, fci_b2, fc_W1, fc_b1, fc_W2, fc_b2, fc_W3, fc_b3):
    raise NotImplementedError("write your pallas kernel here")


# speedup vs baseline: 47.6922x; 47.6922x over previous
import jax
import jax.numpy as jnp
from jax.experimental import pallas as pl
from jax.experimental.pallas import tpu as pltpu

G, N, F, EMB, R, L, E = 128, 60, 128, 128, 4, 2, 1024
NP = 64


def _gcn_body(x_ref, ei_ref, ew_ref, gw_ref, gb_ref, w1_ref, b1_ref,
              w2_ref, b2_ref, out_ref):
    f32 = jnp.float32
    x = x_ref[0]
    xp = jnp.concatenate([x, jnp.zeros((NP - N, F), f32)], axis=0)

    lane_iota = jax.lax.broadcasted_iota(jnp.int32, (E, NP), 1)
    eye = jnp.where(
        jax.lax.broadcasted_iota(jnp.int32, (NP, NP), 0)
        == jax.lax.broadcasted_iota(jnp.int32, (NP, NP), 1), 1.0, 0.0)

    adjs = []
    dinvs = []
    for j in range(R):
        ej = ei_ref[0, j]
        ew = ew_ref[0, j]
        row = ej[:, 0:1]
        col = ej[:, 1:2]
        row_oh = jnp.where(lane_iota == row, 1.0, 0.0)
        col_oh = jnp.where(lane_iota == col, 1.0, 0.0)
        adj = jax.lax.dot_general(col_oh * ew, row_oh,
                                  (((0,), (0,)), ((), ())),
                                  preferred_element_type=f32)
        deg = jnp.sum(adj, axis=1, keepdims=True) + 1.0
        dinv = jnp.where(deg > 0, jax.lax.rsqrt(jnp.where(deg > 0, deg, 1.0)),
                         0.0)
        adjs.append(adj + eye)
        dinvs.append(dinv)

    cur = xp
    for l in range(L):
        wcat = jnp.concatenate([gw_ref[l, j] for j in range(R)], axis=1)
        h_all = jnp.dot(cur, wcat, preferred_element_type=f32)
        outs = []
        for j in range(R):
            hj = h_all[:, j * EMB:(j + 1) * EMB]
            oj = dinvs[j] * jnp.dot(adjs[j], dinvs[j] * hj,
                                    preferred_element_type=f32)
            outs.append(oj + gb_ref[l, j:j + 1, :])
        acc = jnp.concatenate(outs, axis=1)
        t = jnp.maximum(jnp.dot(acc, w1_ref[...],
                                preferred_element_type=f32) + b1_ref[...], 0.0)
        cur = jnp.dot(t, w2_ref[...], preferred_element_type=f32) + b2_ref[...]

    out_ref[0] = cur[:N, :]


def _fc_body(flat_ref, w1_ref, b1_ref, w2_ref, b2_ref, w3_ref, b3_ref,
             out_ref):
    f32 = jnp.float32
    h = jnp.maximum(jnp.dot(flat_ref[...], w1_ref[...],
                            preferred_element_type=f32) + b1_ref[...], 0.0)
    h = jnp.maximum(jnp.dot(h, w2_ref[...],
                            preferred_element_type=f32) + b2_ref[...], 0.0)
    out_ref[...] = jnp.dot(h, w3_ref[...],
                           preferred_element_type=f32) + b3_ref[...]


def kernel(x, edge_index, edge_weight, gcn_W, gcn_b, fci_W1, fci_b1, fci_W2,
           fci_b2, fc_W1, fc_b1, fc_W2, fc_b2, fc_W3, fc_b3):
    ei_t = jnp.transpose(edge_index, (0, 1, 3, 2))
    ew_r = edge_weight[..., None]
    b1 = fci_b1.reshape(1, EMB)
    b2 = fci_b2.reshape(1, EMB)

    cur = pl.pallas_call(
        _gcn_body,
        grid=(G,),
        in_specs=[
            pl.BlockSpec((1, N, F), lambda g: (g, 0, 0)),
            pl.BlockSpec((1, R, E, 2), lambda g: (g, 0, 0, 0)),
            pl.BlockSpec((1, R, E, 1), lambda g: (g, 0, 0, 0)),
            pl.BlockSpec((L, R, F, EMB), lambda g: (0, 0, 0, 0)),
            pl.BlockSpec((L, R, EMB), lambda g: (0, 0, 0)),
            pl.BlockSpec((R * EMB, EMB), lambda g: (0, 0)),
            pl.BlockSpec((1, EMB), lambda g: (0, 0)),
            pl.BlockSpec((EMB, EMB), lambda g: (0, 0)),
            pl.BlockSpec((1, EMB), lambda g: (0, 0)),
        ],
        out_specs=pl.BlockSpec((1, N, EMB), lambda g: (g, 0, 0)),
        out_shape=jax.ShapeDtypeStruct((G, N, EMB), jnp.float32),
    )(x, ei_t, ew_r, gcn_W, gcn_b, fci_W1, b1, fci_W2, b2)

    flat = cur.reshape(G, N * EMB)
    out = pl.pallas_call(
        _fc_body,
        in_specs=[
            pl.BlockSpec((G, N * EMB), lambda: (0, 0)),
            pl.BlockSpec((N * EMB, EMB), lambda: (0, 0)),
            pl.BlockSpec((1, EMB), lambda: (0, 0)),
            pl.BlockSpec((EMB, EMB), lambda: (0, 0)),
            pl.BlockSpec((1, EMB), lambda: (0, 0)),
            pl.BlockSpec((EMB, 2), lambda: (0, 0)),
            pl.BlockSpec((1, 2), lambda: (0, 0)),
        ],
        out_specs=pl.BlockSpec((G, 2), lambda: (0, 0)),
        out_shape=jax.ShapeDtypeStruct((G, 2), jnp.float32),
    )(flat, fc_W1, fc_b1.reshape(1, EMB), fc_W2, fc_b2.reshape(1, EMB),
      fc_W3, fc_b3.reshape(1, 2))
    return out


# speedup vs baseline: 138.5002x; 2.9040x over previous
import functools

import jax
import jax.numpy as jnp
from jax import lax
from jax.experimental import pallas as pl
from jax.experimental.pallas import tpu as pltpu
from jax.experimental.pallas import tpu_sc as plsc

G, N, F, EMB, R, L, E = 128, 60, 128, 128, 4, 2, 1024
NP = 64
NW = 32
TASKS_PER_W = (G * R) // NW


def _sc_adj_body(ei_hbm, ew_hbm, out_hbm, row_v, col_v, ew_v, a_v):
    wid = lax.axis_index("s") * 2 + lax.axis_index("c")

    def task_body(t, carry):
        g = t // R
        r = t - g * R
        pltpu.sync_copy(ei_hbm.at[g, r, 0], row_v)
        pltpu.sync_copy(ei_hbm.at[g, r, 1], col_v)
        pltpu.sync_copy(ew_hbm.at[g, r], ew_v)

        def zero_body(i, c):
            a_v[pl.ds(i * 16, 16)] = jnp.zeros((16,), jnp.float32)
            return c

        lax.fori_loop(0, (NP * NP) // 16, zero_body, 0)

        def edge_body(i, c):
            c16 = col_v[pl.ds(i * 16, 16)]
            r16 = row_v[pl.ds(i * 16, 16)]
            w16 = ew_v[pl.ds(i * 16, 16)]
            plsc.addupdate_scatter(a_v, [c16 * NP + r16], w16)
            return c

        lax.fori_loop(0, E // 16, edge_body, 0)
        pltpu.sync_copy(a_v, out_hbm.at[g, r])
        return carry

    lax.fori_loop(wid * TASKS_PER_W, (wid + 1) * TASKS_PER_W, task_body, 0)


_sc_build_adj = functools.partial(
    pl.kernel,
    out_type=jax.ShapeDtypeStruct((G, R, NP * NP), jnp.float32),
    scratch_types=[
        pltpu.VMEM((E,), jnp.int32),
        pltpu.VMEM((E,), jnp.int32),
        pltpu.VMEM((E,), jnp.float32),
        pltpu.VMEM((NP * NP,), jnp.float32),
    ],
    mesh=plsc.VectorSubcoreMesh(core_axis_name="c", subcore_axis_name="s"),
    compiler_params=pltpu.CompilerParams(needs_layout_passes=False),
)(_sc_adj_body)


def _gcn_body(x_ref, adj_ref, gw_ref, gb_ref, w1_ref, b1_ref,
              w2_ref, b2_ref, out_ref):
    f32 = jnp.float32
    x = x_ref[0]
    xp = jnp.concatenate([x, jnp.zeros((NP - N, F), f32)], axis=0)

    eye = jnp.where(
        jax.lax.broadcasted_iota(jnp.int32, (NP, NP), 0)
        == jax.lax.broadcasted_iota(jnp.int32, (NP, NP), 1), 1.0, 0.0)

    adjs = []
    dinvs = []
    for j in range(R):
        adj = adj_ref[0, j]
        deg = jnp.sum(adj, axis=1, keepdims=True) + 1.0
        dinv = jnp.where(deg > 0, jax.lax.rsqrt(jnp.where(deg > 0, deg, 1.0)),
                         0.0)
        adjs.append(adj + eye)
        dinvs.append(dinv)

    cur = xp
    for l in range(L):
        wcat = jnp.concatenate([gw_ref[l, j] for j in range(R)], axis=1)
        h_all = jnp.dot(cur, wcat, preferred_element_type=f32)
        outs = []
        for j in range(R):
            hj = h_all[:, j * EMB:(j + 1) * EMB]
            oj = dinvs[j] * jnp.dot(adjs[j], dinvs[j] * hj,
                                    preferred_element_type=f32)
            outs.append(oj + gb_ref[l, j:j + 1, :])
        acc = jnp.concatenate(outs, axis=1)
        t = jnp.maximum(jnp.dot(acc, w1_ref[...],
                                preferred_element_type=f32) + b1_ref[...], 0.0)
        cur = jnp.dot(t, w2_ref[...], preferred_element_type=f32) + b2_ref[...]

    out_ref[0] = cur[:N, :]


def _fc_body(flat_ref, w1_ref, b1_ref, w2_ref, b2_ref, w3_ref, b3_ref,
             out_ref):
    f32 = jnp.float32
    h = jnp.maximum(jnp.dot(flat_ref[...], w1_ref[...],
                            preferred_element_type=f32) + b1_ref[...], 0.0)
    h = jnp.maximum(jnp.dot(h, w2_ref[...],
                            preferred_element_type=f32) + b2_ref[...], 0.0)
    out_ref[...] = jnp.dot(h, w3_ref[...],
                           preferred_element_type=f32) + b3_ref[...]


def kernel(x, edge_index, edge_weight, gcn_W, gcn_b, fci_W1, fci_b1, fci_W2,
           fci_b2, fc_W1, fc_b1, fc_W2, fc_b2, fc_W3, fc_b3):
    adj = _sc_build_adj(edge_index, edge_weight).reshape(G, R, NP, NP)
    b1 = fci_b1.reshape(1, EMB)
    b2 = fci_b2.reshape(1, EMB)

    cur = pl.pallas_call(
        _gcn_body,
        grid=(G,),
        in_specs=[
            pl.BlockSpec((1, N, F), lambda g: (g, 0, 0)),
            pl.BlockSpec((1, R, NP, NP), lambda g: (g, 0, 0, 0)),
            pl.BlockSpec((L, R, F, EMB), lambda g: (0, 0, 0, 0)),
            pl.BlockSpec((L, R, EMB), lambda g: (0, 0, 0)),
            pl.BlockSpec((R * EMB, EMB), lambda g: (0, 0)),
            pl.BlockSpec((1, EMB), lambda g: (0, 0)),
            pl.BlockSpec((EMB, EMB), lambda g: (0, 0)),
            pl.BlockSpec((1, EMB), lambda g: (0, 0)),
        ],
        out_specs=pl.BlockSpec((1, N, EMB), lambda g: (g, 0, 0)),
        out_shape=jax.ShapeDtypeStruct((G, N, EMB), jnp.float32),
    )(x, adj, gcn_W, gcn_b, fci_W1, b1, fci_W2, b2)

    flat = cur.reshape(G, N * EMB)
    out = pl.pallas_call(
        _fc_body,
        in_specs=[
            pl.BlockSpec((G, N * EMB), lambda: (0, 0)),
            pl.BlockSpec((N * EMB, EMB), lambda: (0, 0)),
            pl.BlockSpec((1, EMB), lambda: (0, 0)),
            pl.BlockSpec((EMB, EMB), lambda: (0, 0)),
            pl.BlockSpec((1, EMB), lambda: (0, 0)),
            pl.BlockSpec((EMB, 2), lambda: (0, 0)),
            pl.BlockSpec((1, 2), lambda: (0, 0)),
        ],
        out_specs=pl.BlockSpec((G, 2), lambda: (0, 0)),
        out_shape=jax.ShapeDtypeStruct((G, 2), jnp.float32),
    )(flat, fc_W1, fc_b1.reshape(1, EMB), fc_W2, fc_b2.reshape(1, EMB),
      fc_W3, fc_b3.reshape(1, 2))
    return out


# speedup vs baseline: 180.5469x; 1.3036x over previous
import functools

import jax
import jax.numpy as jnp
from jax import lax
from jax.experimental import pallas as pl
from jax.experimental.pallas import tpu as pltpu
from jax.experimental.pallas import tpu_sc as plsc

G, N, F, EMB, R, L, E = 128, 60, 128, 128, 4, 2, 1024
NP = 64
NW = 32
GPW = G // NW


def _sc_adj_body(ei_hbm, ew_hbm, out_hbm, ei_v, ew_v, a_v):
    wid = lax.axis_index("s") * 2 + lax.axis_index("c")
    zero16 = jnp.zeros((16,), jnp.float32)

    def graph_body(gi, carry):
        g = wid * GPW + gi
        pltpu.sync_copy(ei_hbm.at[g], ei_v)
        pltpu.sync_copy(ew_hbm.at[g], ew_v)

        def zero_body(i, c):
            for r in range(R):
                for q in range(NP // 16):
                    a_v[r, i, pl.ds(q * 16, 16)] = zero16
            return c

        lax.fori_loop(0, NP, zero_body, 0)

        def edge_body(i, c):
            for r in range(R):
                r16 = ei_v[r, 0, pl.ds(i * 16, 16)]
                c16 = ei_v[r, 1, pl.ds(i * 16, 16)]
                w16 = ew_v[r, pl.ds(i * 16, 16)]
                plsc.addupdate_scatter(
                    a_v, [jnp.full((16,), r, jnp.int32), c16, r16], w16)
            return c

        lax.fori_loop(0, E // 16, edge_body, 0)
        pltpu.sync_copy(a_v, out_hbm.at[g])
        return carry

    lax.fori_loop(0, GPW, graph_body, 0)


_sc_build_adj = functools.partial(
    pl.kernel,
    out_type=jax.ShapeDtypeStruct((G, R, NP, NP), jnp.float32),
    scratch_types=[
        pltpu.VMEM((R, 2, E), jnp.int32),
        pltpu.VMEM((R, E), jnp.float32),
        pltpu.VMEM((R, NP, NP), jnp.float32),
    ],
    mesh=plsc.VectorSubcoreMesh(core_axis_name="c", subcore_axis_name="s"),
    compiler_params=pltpu.CompilerParams(needs_layout_passes=False),
)(_sc_adj_body)


def _gcn_body(x_ref, adj_ref, gw_ref, gb_ref, w1_ref, b1_ref,
              w2_ref, b2_ref, out_ref):
    f32 = jnp.float32
    bf16 = jnp.bfloat16
    x = x_ref[0]
    xp = jnp.concatenate([x, jnp.zeros((NP - N, F), f32)], axis=0)

    eye = jnp.where(
        jax.lax.broadcasted_iota(jnp.int32, (NP, NP), 0)
        == jax.lax.broadcasted_iota(jnp.int32, (NP, NP), 1), 1.0, 0.0)

    adjs = []
    dinvs = []
    for j in range(R):
        adj = adj_ref[0, j]
        deg = jnp.sum(adj, axis=1, keepdims=True) + 1.0
        dinv = jnp.where(deg > 0, jax.lax.rsqrt(jnp.where(deg > 0, deg, 1.0)),
                         0.0)
        adjs.append((adj + eye).astype(bf16))
        dinvs.append(dinv)

    cur = xp
    for l in range(L):
        wcat = jnp.concatenate([gw_ref[l, j] for j in range(R)], axis=1)
        h_all = jnp.dot(cur, wcat, preferred_element_type=f32)
        outs = []
        for j in range(R):
            hj = h_all[:, j * EMB:(j + 1) * EMB]
            oj = dinvs[j] * jnp.dot(adjs[j], (dinvs[j] * hj).astype(bf16),
                                    preferred_element_type=f32)
            outs.append(oj + gb_ref[l, j:j + 1, :])
        acc = jnp.concatenate(outs, axis=1)
        t = jnp.maximum(jnp.dot(acc, w1_ref[...],
                                preferred_element_type=f32) + b1_ref[...], 0.0)
        cur = jnp.dot(t, w2_ref[...], preferred_element_type=f32) + b2_ref[...]

    out_ref[0] = cur[:N, :]


def _fc_body(flat_ref, w1_ref, b1_ref, w2_ref, b2_ref, w3_ref, b3_ref,
             out_ref):
    f32 = jnp.float32
    h = jnp.maximum(jnp.dot(flat_ref[...], w1_ref[...],
                            preferred_element_type=f32) + b1_ref[...], 0.0)
    h = jnp.maximum(jnp.dot(h, w2_ref[...],
                            preferred_element_type=f32) + b2_ref[...], 0.0)
    out_ref[...] = jnp.dot(h, w3_ref[...],
                           preferred_element_type=f32) + b3_ref[...]


def kernel(x, edge_index, edge_weight, gcn_W, gcn_b, fci_W1, fci_b1, fci_W2,
           fci_b2, fc_W1, fc_b1, fc_W2, fc_b2, fc_W3, fc_b3):
    adj = _sc_build_adj(edge_index, edge_weight)
    b1 = fci_b1.reshape(1, EMB)
    b2 = fci_b2.reshape(1, EMB)

    cur = pl.pallas_call(
        _gcn_body,
        grid=(G,),
        in_specs=[
            pl.BlockSpec((1, N, F), lambda g: (g, 0, 0)),
            pl.BlockSpec((1, R, NP, NP), lambda g: (g, 0, 0, 0)),
            pl.BlockSpec((L, R, F, EMB), lambda g: (0, 0, 0, 0)),
            pl.BlockSpec((L, R, EMB), lambda g: (0, 0, 0)),
            pl.BlockSpec((R * EMB, EMB), lambda g: (0, 0)),
            pl.BlockSpec((1, EMB), lambda g: (0, 0)),
            pl.BlockSpec((EMB, EMB), lambda g: (0, 0)),
            pl.BlockSpec((1, EMB), lambda g: (0, 0)),
        ],
        out_specs=pl.BlockSpec((1, N, EMB), lambda g: (g, 0, 0)),
        out_shape=jax.ShapeDtypeStruct((G, N, EMB), jnp.float32),
    )(x, adj, gcn_W, gcn_b, fci_W1, b1, fci_W2, b2)

    flat = cur.reshape(G, N * EMB)
    out = pl.pallas_call(
        _fc_body,
        in_specs=[
            pl.BlockSpec((G, N * EMB), lambda: (0, 0)),
            pl.BlockSpec((N * EMB, EMB), lambda: (0, 0)),
            pl.BlockSpec((1, EMB), lambda: (0, 0)),
            pl.BlockSpec((EMB, EMB), lambda: (0, 0)),
            pl.BlockSpec((1, EMB), lambda: (0, 0)),
            pl.BlockSpec((EMB, 2), lambda: (0, 0)),
            pl.BlockSpec((1, 2), lambda: (0, 0)),
        ],
        out_specs=pl.BlockSpec((G, 2), lambda: (0, 0)),
        out_shape=jax.ShapeDtypeStruct((G, 2), jnp.float32),
    )(flat, fc_W1, fc_b1.reshape(1, EMB), fc_W2, fc_b2.reshape(1, EMB),
      fc_W3, fc_b3.reshape(1, 2))
    return out


# speedup vs baseline: 346.3464x; 1.9183x over previous
import functools

import jax
import jax.numpy as jnp
from jax import lax
from jax.experimental import pallas as pl
from jax.experimental.pallas import tpu as pltpu
from jax.experimental.pallas import tpu_sc as plsc

G, N, F, EMB, R, L, E = 128, 60, 128, 128, 4, 2, 1024
NP = 64
NW = 32
GPW = G // NW


def _sc_adj_body(ei_hbm, ew_hbm, out_hbm, ei_v, ew_v, a_v):
    wid = lax.axis_index("s") * 2 + lax.axis_index("c")
    zero16 = jnp.zeros((16,), jnp.float32)

    def graph_body(gi, carry):
        g = wid * GPW + gi
        pltpu.sync_copy(ei_hbm.at[g], ei_v)
        pltpu.sync_copy(ew_hbm.at[g], ew_v)

        def zero_body(i, c):
            for r in range(R):
                for q in range(NP // 16):
                    a_v[r, i, pl.ds(q * 16, 16)] = zero16
            return c

        lax.fori_loop(0, NP, zero_body, 0)

        def edge_body(i, c):
            for r in range(R):
                r16 = ei_v[r, 0, pl.ds(i * 16, 16)]
                c16 = ei_v[r, 1, pl.ds(i * 16, 16)]
                w16 = ew_v[r, pl.ds(i * 16, 16)]
                plsc.addupdate_scatter(
                    a_v, [jnp.full((16,), r, jnp.int32), c16, r16], w16)
            return c

        lax.fori_loop(0, E // 16, edge_body, 0)
        pltpu.sync_copy(a_v, out_hbm.at[g])
        return carry

    lax.fori_loop(0, GPW, graph_body, 0)


_sc_build_adj = functools.partial(
    pl.kernel,
    out_type=jax.ShapeDtypeStruct((G, R, NP, NP), jnp.float32),
    scratch_types=[
        pltpu.VMEM((R, 2, E), jnp.int32),
        pltpu.VMEM((R, E), jnp.float32),
        pltpu.VMEM((R, NP, NP), jnp.float32),
    ],
    mesh=plsc.VectorSubcoreMesh(core_axis_name="c", subcore_axis_name="s"),
    compiler_params=pltpu.CompilerParams(needs_layout_passes=False),
)(_sc_adj_body)


GB = 4


def _gcn_body(x_ref, adj_ref, gw_ref, gb_ref, w1_ref, b1_ref,
              w2_ref, b2_ref, out_ref):
    f32 = jnp.float32
    bf16 = jnp.bfloat16
    zpad = jnp.zeros((NP - N, F), f32)
    xp = jnp.concatenate(
        sum([[x_ref[i], zpad] for i in range(GB)], []), axis=0)

    eye = jnp.where(
        jax.lax.broadcasted_iota(jnp.int32, (NP, NP), 0)
        == jax.lax.broadcasted_iota(jnp.int32, (NP, NP), 1), 1.0, 0.0)

    adjs = {}
    dinvs = {}
    for i in range(GB):
        for j in range(R):
            adj = adj_ref[i, j]
            deg = jnp.sum(adj, axis=1, keepdims=True) + 1.0
            dinv = jnp.where(deg > 0,
                             jax.lax.rsqrt(jnp.where(deg > 0, deg, 1.0)), 0.0)
            adjs[i, j] = (adj + eye).astype(bf16)
            dinvs[i, j] = dinv

    cur = xp
    for l in range(L):
        wcat = jnp.concatenate([gw_ref[l, j] for j in range(R)], axis=1)
        h_all = jnp.dot(cur, wcat, preferred_element_type=f32)
        rows = []
        for i in range(GB):
            outs = []
            for j in range(R):
                hj = h_all[i * NP:(i + 1) * NP, j * EMB:(j + 1) * EMB]
                oj = dinvs[i, j] * jnp.dot(
                    adjs[i, j], (dinvs[i, j] * hj).astype(bf16),
                    preferred_element_type=f32)
                outs.append(oj + gb_ref[l, j:j + 1, :])
            rows.append(jnp.concatenate(outs, axis=1))
        acc = jnp.concatenate(rows, axis=0)
        t = jnp.maximum(jnp.dot(acc, w1_ref[...],
                                preferred_element_type=f32) + b1_ref[...], 0.0)
        cur = jnp.dot(t, w2_ref[...], preferred_element_type=f32) + b2_ref[...]

    for i in range(GB):
        out_ref[i] = cur[i * NP:i * NP + N, :]


def _fc_body(flat_ref, w1_ref, b1_ref, w2_ref, b2_ref, w3_ref, b3_ref,
             out_ref):
    f32 = jnp.float32
    h = jnp.maximum(jnp.dot(flat_ref[...], w1_ref[...],
                            preferred_element_type=f32) + b1_ref[...], 0.0)
    h = jnp.maximum(jnp.dot(h, w2_ref[...],
                            preferred_element_type=f32) + b2_ref[...], 0.0)
    out_ref[...] = jnp.dot(h, w3_ref[...],
                           preferred_element_type=f32) + b3_ref[...]


def kernel(x, edge_index, edge_weight, gcn_W, gcn_b, fci_W1, fci_b1, fci_W2,
           fci_b2, fc_W1, fc_b1, fc_W2, fc_b2, fc_W3, fc_b3):
    adj = _sc_build_adj(edge_index, edge_weight)
    b1 = fci_b1.reshape(1, EMB)
    b2 = fci_b2.reshape(1, EMB)

    cur = pl.pallas_call(
        _gcn_body,
        grid=(G // GB,),
        in_specs=[
            pl.BlockSpec((GB, N, F), lambda g: (g, 0, 0)),
            pl.BlockSpec((GB, R, NP, NP), lambda g: (g, 0, 0, 0)),
            pl.BlockSpec((L, R, F, EMB), lambda g: (0, 0, 0, 0)),
            pl.BlockSpec((L, R, EMB), lambda g: (0, 0, 0)),
            pl.BlockSpec((R * EMB, EMB), lambda g: (0, 0)),
            pl.BlockSpec((1, EMB), lambda g: (0, 0)),
            pl.BlockSpec((EMB, EMB), lambda g: (0, 0)),
            pl.BlockSpec((1, EMB), lambda g: (0, 0)),
        ],
        out_specs=pl.BlockSpec((GB, N, EMB), lambda g: (g, 0, 0)),
        out_shape=jax.ShapeDtypeStruct((G, N, EMB), jnp.float32),
    )(x, adj, gcn_W, gcn_b, fci_W1, b1, fci_W2, b2)

    flat = cur.reshape(G, N * EMB)
    out = pl.pallas_call(
        _fc_body,
        in_specs=[
            pl.BlockSpec((G, N * EMB), lambda: (0, 0)),
            pl.BlockSpec((N * EMB, EMB), lambda: (0, 0)),
            pl.BlockSpec((1, EMB), lambda: (0, 0)),
            pl.BlockSpec((EMB, EMB), lambda: (0, 0)),
            pl.BlockSpec((1, EMB), lambda: (0, 0)),
            pl.BlockSpec((EMB, 2), lambda: (0, 0)),
            pl.BlockSpec((1, 2), lambda: (0, 0)),
        ],
        out_specs=pl.BlockSpec((G, 2), lambda: (0, 0)),
        out_shape=jax.ShapeDtypeStruct((G, 2), jnp.float32),
    )(flat, fc_W1, fc_b1.reshape(1, EMB), fc_W2, fc_b2.reshape(1, EMB),
      fc_W3, fc_b3.reshape(1, 2))
    return out


# speedup vs baseline: 365.4535x; 1.0552x over previous
import functools

import jax
import jax.numpy as jnp
from jax import lax
from jax.experimental import pallas as pl
from jax.experimental.pallas import tpu as pltpu
from jax.experimental.pallas import tpu_sc as plsc

G, N, F, EMB, R, L, E = 128, 60, 128, 128, 4, 2, 1024
NP = 64
NW = 32
GPW = G // NW


def _sc_adj_body(ei_hbm, ew_hbm, out_hbm, ei_v, ew_v, a_v):
    wid = lax.axis_index("s") * 2 + lax.axis_index("c")
    zero16 = jnp.zeros((16,), jnp.float32)

    def graph_body(gi, carry):
        g = wid * GPW + gi
        pltpu.sync_copy(ei_hbm.at[g], ei_v)
        pltpu.sync_copy(ew_hbm.at[g], ew_v)

        def zero_body(i, c):
            for r in range(R):
                for q in range(NP // 16):
                    a_v[r, i, pl.ds(q * 16, 16)] = zero16
            return c

        lax.fori_loop(0, NP, zero_body, 0)

        def edge_body(i, c):
            for r in range(R):
                r16 = ei_v[r, 0, pl.ds(i * 16, 16)]
                c16 = ei_v[r, 1, pl.ds(i * 16, 16)]
                w16 = ew_v[r, pl.ds(i * 16, 16)]
                plsc.addupdate_scatter(
                    a_v, [jnp.full((16,), r, jnp.int32), c16, r16], w16)
            return c

        lax.fori_loop(0, E // 16, edge_body, 0)
        pltpu.sync_copy(a_v, out_hbm.at[g])
        return carry

    lax.fori_loop(0, GPW, graph_body, 0)


_sc_build_adj = functools.partial(
    pl.kernel,
    out_type=jax.ShapeDtypeStruct((G, R, NP, NP), jnp.float32),
    scratch_types=[
        pltpu.VMEM((R, 2, E), jnp.int32),
        pltpu.VMEM((R, E), jnp.float32),
        pltpu.VMEM((R, NP, NP), jnp.float32),
    ],
    mesh=plsc.VectorSubcoreMesh(core_axis_name="c", subcore_axis_name="s"),
    compiler_params=pltpu.CompilerParams(needs_layout_passes=False),
)(_sc_adj_body)


GB = 4


def _gcn_body(x_ref, adj_ref, gw_ref, gb_ref, w1_ref, b1_ref,
              w2_ref, b2_ref, out_ref):
    f32 = jnp.float32
    bf16 = jnp.bfloat16
    zpad = jnp.zeros((NP - N, F), f32)
    xp = jnp.concatenate(
        sum([[x_ref[i], zpad] for i in range(GB)], []), axis=0)

    eye = jnp.where(
        jax.lax.broadcasted_iota(jnp.int32, (NP, NP), 0)
        == jax.lax.broadcasted_iota(jnp.int32, (NP, NP), 1), 1.0, 0.0)

    adjs = {}
    dinvs = {}
    for i in range(GB):
        for j in range(R):
            adj = adj_ref[i, j]
            deg = jnp.sum(adj, axis=1, keepdims=True) + 1.0
            dinv = jnp.where(deg > 0,
                             jax.lax.rsqrt(jnp.where(deg > 0, deg, 1.0)), 0.0)
            adjs[i, j] = (adj + eye).astype(bf16)
            dinvs[i, j] = dinv

    cur = xp
    for l in range(L):
        wcat = jnp.concatenate([gw_ref[l, j] for j in range(R)], axis=1)
        h_all = jnp.dot(cur, wcat, preferred_element_type=f32)
        rows = []
        for i in range(GB):
            outs = []
            for j in range(R):
                hj = h_all[i * NP:(i + 1) * NP, j * EMB:(j + 1) * EMB]
                oj = dinvs[i, j] * jnp.dot(
                    adjs[i, j], (dinvs[i, j] * hj).astype(bf16),
                    preferred_element_type=f32)
                outs.append(oj + gb_ref[l, j:j + 1, :])
            rows.append(jnp.concatenate(outs, axis=1))
        acc = jnp.concatenate(rows, axis=0)
        t = jnp.maximum(jnp.dot(acc, w1_ref[...],
                                preferred_element_type=f32) + b1_ref[...], 0.0)
        cur = jnp.dot(t, w2_ref[...], preferred_element_type=f32) + b2_ref[...]

    for i in range(GB):
        out_ref[i] = cur[i * NP:i * NP + N, :]


def _fc_body(cur_ref, w1_ref, b1_ref, w2_ref, b2_ref, w3_ref, b3_ref,
             out_ref):
    f32 = jnp.float32
    h = b1_ref[...]
    for n in range(N):
        h = h + jnp.dot(cur_ref[:, n, :], w1_ref[n],
                        preferred_element_type=f32)
    h = jnp.maximum(h, 0.0)
    h = jnp.maximum(jnp.dot(h, w2_ref[...],
                            preferred_element_type=f32) + b2_ref[...], 0.0)
    out_ref[...] = jnp.dot(h, w3_ref[...],
                           preferred_element_type=f32) + b3_ref[...]


def kernel(x, edge_index, edge_weight, gcn_W, gcn_b, fci_W1, fci_b1, fci_W2,
           fci_b2, fc_W1, fc_b1, fc_W2, fc_b2, fc_W3, fc_b3):
    adj = _sc_build_adj(edge_index, edge_weight)
    b1 = fci_b1.reshape(1, EMB)
    b2 = fci_b2.reshape(1, EMB)

    cur = pl.pallas_call(
        _gcn_body,
        grid=(G // GB,),
        in_specs=[
            pl.BlockSpec((GB, N, F), lambda g: (g, 0, 0)),
            pl.BlockSpec((GB, R, NP, NP), lambda g: (g, 0, 0, 0)),
            pl.BlockSpec((L, R, F, EMB), lambda g: (0, 0, 0, 0)),
            pl.BlockSpec((L, R, EMB), lambda g: (0, 0, 0)),
            pl.BlockSpec((R * EMB, EMB), lambda g: (0, 0)),
            pl.BlockSpec((1, EMB), lambda g: (0, 0)),
            pl.BlockSpec((EMB, EMB), lambda g: (0, 0)),
            pl.BlockSpec((1, EMB), lambda g: (0, 0)),
        ],
        out_specs=pl.BlockSpec((GB, N, EMB), lambda g: (g, 0, 0)),
        out_shape=jax.ShapeDtypeStruct((G, N, EMB), jnp.float32),
    )(x, adj, gcn_W, gcn_b, fci_W1, b1, fci_W2, b2)

    w1r = fc_W1.reshape(N, EMB, EMB)
    out = pl.pallas_call(
        _fc_body,
        in_specs=[
            pl.BlockSpec((G, N, EMB), lambda: (0, 0, 0)),
            pl.BlockSpec((N, EMB, EMB), lambda: (0, 0, 0)),
            pl.BlockSpec((1, EMB), lambda: (0, 0)),
            pl.BlockSpec((EMB, EMB), lambda: (0, 0)),
            pl.BlockSpec((1, EMB), lambda: (0, 0)),
            pl.BlockSpec((EMB, 2), lambda: (0, 0)),
            pl.BlockSpec((1, 2), lambda: (0, 0)),
        ],
        out_specs=pl.BlockSpec((G, 2), lambda: (0, 0)),
        out_shape=jax.ShapeDtypeStruct((G, 2), jnp.float32),
    )(cur, w1r, fc_b1.reshape(1, EMB), fc_W2, fc_b2.reshape(1, EMB),
      fc_W3, fc_b3.reshape(1, 2))
    return out
